# Initial kernel scaffold; baseline (speedup 1.0000x reference)
import jax
import jax.numpy as jnp
from jax.experimental import pallas as pl


def kernel(z, codebook):
    raise NotImplementedError("write your pallas kernel here")


# speedup vs baseline: 1.4082x; 1.4082x over previous
import jax
import jax.numpy as jnp
from jax.experimental import pallas as pl
from jax.experimental.pallas import tpu as pltpu

_NUM_CODES = 1024
_CODE_DIM = 64
_COMMIT = 0.25


def _vq_body(z_ref, cb_ref, zq_ref, codes_ref, loss_ref):
    b = pl.program_id(0)
    nb = pl.num_programs(0)
    z_b = z_ref[0]
    cb = cb_ref[...]

    m = jax.lax.dot_general(cb, z_b, (((1,), (0,)), ((), ())),
                            preferred_element_type=jnp.float32)
    x2 = jnp.sum(z_b * z_b, axis=0, keepdims=True)
    e2 = jnp.sum(cb * cb, axis=1, keepdims=True)
    dist = (x2 + e2) - 2.0 * m

    minval = jnp.min(dist, axis=0, keepdims=True)
    iota_c = jax.lax.broadcasted_iota(jnp.int32, (_NUM_CODES, dist.shape[1]), 0)
    masked = jnp.where(dist == minval, iota_c, _NUM_CODES)
    codes = jnp.min(masked, axis=0, keepdims=True)
    codes_ref[0] = codes

    one_hot = (iota_c == codes).astype(jnp.float32)
    zq_b = jax.lax.dot_general(cb, one_hot, (((0,), (0,)), ((), ())),
                               precision=jax.lax.Precision.HIGHEST,
                               preferred_element_type=jnp.float32)
    zq_ref[0] = z_b + (zq_b - z_b)

    diff = z_b - zq_b
    partial = jnp.sum(diff * diff)
    prev = jnp.where(b == 0, 0.0, loss_ref[0, 0])
    acc = prev + partial
    scale = _COMMIT / (nb * _CODE_DIM * zq_b.shape[1])
    loss_ref[0, 0] = jnp.where(b == nb - 1, acc * scale, acc)


def kernel(z, codebook):
    B, C, T = z.shape
    zq, codes3, loss = pl.pallas_call(
        _vq_body,
        grid=(B,),
        in_specs=[
            pl.BlockSpec((1, C, T), lambda b: (b, 0, 0)),
            pl.BlockSpec((_NUM_CODES, C), lambda b: (0, 0)),
        ],
        out_specs=[
            pl.BlockSpec((1, C, T), lambda b: (b, 0, 0)),
            pl.BlockSpec((1, 1, T), lambda b: (b, 0, 0)),
            pl.BlockSpec((1, 1), lambda b: (0, 0), memory_space=pltpu.SMEM),
        ],
        out_shape=[
            jax.ShapeDtypeStruct((B, C, T), jnp.float32),
            jax.ShapeDtypeStruct((B, 1, T), jnp.int32),
            jax.ShapeDtypeStruct((1, 1), jnp.float32),
        ],
        compiler_params=pltpu.CompilerParams(
            dimension_semantics=("arbitrary",)),
    )(z, codebook)
    return zq, codes3.reshape(B, T), loss[0, 0]
